# baseline (speedup 1.0000x reference)
import functools

import jax
import jax.numpy as jnp
from jax import lax
from jax.experimental import pallas as pl
from jax.experimental.pallas import tpu as pltpu
from jax.experimental.pallas import tpu_sc as plsc

_TARGET = 0
_TEST = 0.0

_ROWS = 32768
_COLS = 256
_NC = 2
_NS = 16
_L = 16
_NW = _NC * _NS
_ROWS_PER_W = _ROWS // _NW
_SLAB = 64
_NSLAB = _ROWS_PER_W // _SLAB


def _join_col0(c0, d0):
    lo = c0 - d0
    hi = c0 + d0
    left = lo <= _TEST
    right = hi > _TEST
    min_hi = jnp.minimum(hi, _TEST)
    cl = (lo + min_hi) * 0.5
    dl = (min_hi - lo) * 0.5
    max_lo = jnp.maximum(lo, _TEST)
    cr = (max_lo + hi) * 0.5
    dr = (hi - max_lo) * 0.5
    both = left & right
    lj = jnp.minimum(cl - dl, cr - dr)
    rj = jnp.maximum(cl + dl, cr + dr)
    cb = (lj + rj) * 0.5
    db = (rj - lj) * 0.5
    new_c0 = jnp.where(both, cb, jnp.where(left, cl, cr))
    new_d0 = jnp.where(both, db, jnp.where(left, dl, dr))
    return new_c0, new_d0


def _sc_body(c_hbm, d_hbm, oc_hbm, od_hbm,
             cb0, db0, cb1, db1, cb2, db2, s0, s1, s2):
    bufs = ((cb0, db0, s0), (cb1, db1, s1), (cb2, db2, s2))
    wid = lax.axis_index("s") * _NC + lax.axis_index("c")
    rbase = wid * _ROWS_PER_W
    iota = lax.iota(jnp.int32, _L)
    zer = jnp.zeros((_L,), jnp.int32)

    def rows(k):
        return pl.ds(rbase + k * _SLAB, _SLAB)

    def fire_in(k):
        cb, db, s = bufs[k % 3]
        pltpu.async_copy(c_hbm.at[rows(k), :], cb, s)
        pltpu.async_copy(d_hbm.at[rows(k), :], db, s)

    def wait_in(k):
        cb, db, s = bufs[k % 3]
        pltpu.make_async_copy(c_hbm.at[rows(k), :], cb, s).wait()
        pltpu.make_async_copy(d_hbm.at[rows(k), :], db, s).wait()

    def fire_out(k):
        cb, db, s = bufs[k % 3]
        pltpu.async_copy(cb, oc_hbm.at[rows(k), :], s)
        pltpu.async_copy(db, od_hbm.at[rows(k), :], s)

    def wait_out(k):
        cb, db, s = bufs[k % 3]
        pltpu.make_async_copy(cb, oc_hbm.at[rows(k), :], s).wait()
        pltpu.make_async_copy(db, od_hbm.at[rows(k), :], s).wait()

    fire_in(0)
    fire_in(1)
    for k in range(_NSLAB):
        cb, db, _ = bufs[k % 3]
        wait_in(k)
        for j in range(_SLAB // _L):
            ridx = iota + (j * _L)
            c0 = plsc.load_gather(cb, [ridx, zer])
            d0 = plsc.load_gather(db, [ridx, zer])
            new_c0, new_d0 = _join_col0(c0, d0)
            plsc.store_scatter(cb, [ridx, zer], new_c0)
            plsc.store_scatter(db, [ridx, zer], new_d0)
        fire_out(k)
        if k + 2 < _NSLAB:
            if k >= 1:
                wait_out(k - 1)
            fire_in(k + 2)
    wait_out(_NSLAB - 3)
    wait_out(_NSLAB - 2)
    wait_out(_NSLAB - 1)


def kernel(c, delta, idx):
    del idx
    mesh = plsc.VectorSubcoreMesh(core_axis_name="c", subcore_axis_name="s")
    f = functools.partial(
        pl.kernel,
        out_type=[
            jax.ShapeDtypeStruct((_ROWS, _COLS), jnp.float32),
            jax.ShapeDtypeStruct((_ROWS, _COLS), jnp.float32),
        ],
        mesh=mesh,
        scratch_types=(
            [pltpu.VMEM((_SLAB, _COLS), jnp.float32)] * 6
            + [pltpu.SemaphoreType.DMA] * 3
        ),
        compiler_params=pltpu.CompilerParams(
            needs_layout_passes=False,
            use_tc_tiling_on_sc=False,
        ),
    )(_sc_body)
    oc, od = f(c, delta)
    return oc, od


# speedup vs baseline: 4.3078x; 4.3078x over previous
import jax
import jax.numpy as jnp
from jax.experimental import pallas as pl
from jax.experimental.pallas import tpu as pltpu

_TARGET = 0
_TEST = 0.0

_ROWS = 32768
_COLS = 256
_BLOCK_ROWS = 4096


def _ifelse_kernel(c_ref, d_ref, oc_ref, od_ref):
    c = c_ref[...]
    d = d_ref[...]
    c0 = c[:, _TARGET:_TARGET + 1]
    d0 = d[:, _TARGET:_TARGET + 1]
    lo = c0 - d0
    hi = c0 + d0
    left = lo <= _TEST
    right = hi > _TEST
    min_hi = jnp.minimum(hi, _TEST)
    cl = (lo + min_hi) * 0.5
    dl = (min_hi - lo) * 0.5
    max_lo = jnp.maximum(lo, _TEST)
    cr = (max_lo + hi) * 0.5
    dr = (hi - max_lo) * 0.5
    both = left & right
    lj = jnp.minimum(cl - dl, cr - dr)
    rj = jnp.maximum(cl + dl, cr + dr)
    cb = (lj + rj) * 0.5
    db = (rj - lj) * 0.5
    new_c0 = jnp.where(both, cb, jnp.where(left, cl, cr))
    new_d0 = jnp.where(both, db, jnp.where(left, dl, dr))
    col = jax.lax.broadcasted_iota(jnp.int32, c.shape, 1)
    is_t = col == _TARGET
    oc_ref[...] = jnp.where(is_t, new_c0, c)
    od_ref[...] = jnp.where(is_t, new_d0, d)


def kernel(c, delta, idx):
    del idx
    spec = pl.BlockSpec((_BLOCK_ROWS, _COLS), lambda i: (i, 0))
    out_c, out_d = pl.pallas_call(
        _ifelse_kernel,
        grid=(_ROWS // _BLOCK_ROWS,),
        in_specs=[spec, spec],
        out_specs=[spec, spec],
        out_shape=[
            jax.ShapeDtypeStruct((_ROWS, _COLS), jnp.float32),
            jax.ShapeDtypeStruct((_ROWS, _COLS), jnp.float32),
        ],
        compiler_params=pltpu.CompilerParams(
            dimension_semantics=("parallel",),
        ),
    )(c, delta)
    return out_c, out_d
